# baseline (speedup 1.0000x reference)
import functools

import jax
import jax.numpy as jnp
from jax import lax
from jax.experimental import pallas as pl
from jax.experimental.pallas import tpu as pltpu
from jax.experimental.pallas import tpu_sc as plsc

_BLOCK_B = 16


def _mse_body(w_ref, x_ref, out_ref, acc_ref):
    i = pl.program_id(0)

    @pl.when(i == 0)
    def _init():
        acc_ref[...] = jnp.zeros_like(acc_ref)

    wt = 127.5 * (jnp.tanh(w_ref[...]) + 1.0)
    d = wt - x_ref[...]
    acc_ref[...] += jnp.sum(d * d, axis=(0, 1))

    @pl.when(i == pl.num_programs(0) - 1)
    def _finish():
        out_ref[0, 0] = jnp.sum(acc_ref[...])


def _mse_sum(w, x):
    b, ch, h, wd = w.shape
    grid = b // _BLOCK_B
    out = pl.pallas_call(
        _mse_body,
        grid=(grid,),
        in_specs=[
            pl.BlockSpec((_BLOCK_B, ch, h, wd), lambda i: (i, 0, 0, 0)),
            pl.BlockSpec((_BLOCK_B, ch, h, wd), lambda i: (i, 0, 0, 0)),
        ],
        out_specs=pl.BlockSpec(memory_space=pltpu.SMEM),
        out_shape=jax.ShapeDtypeStruct((1, 1), jnp.float32),
        scratch_shapes=[pltpu.VMEM((h, wd), jnp.float32)],
        compiler_params=pltpu.CompilerParams(
            dimension_semantics=("arbitrary",),
        ),
    )(w, x)
    return out[0, 0]


def _margin_kernel(n_classes, lg_hbm, tg_hbm, out_hbm, lg_v, tg_v, mg_v, sem):
    wid = lax.axis_index("s") * 2 + lax.axis_index("c")

    @pl.when(wid < 16)
    def _active():
        pltpu.sync_copy(lg_hbm.at[wid], lg_v)
        pltpu.sync_copy(tg_hbm.at[wid], tg_v)
        t = tg_v[...]

        def step(j, carry):
            mx, ts = carry
            val = lg_v[pl.ds(j * 16, 16)]
            m = t == j
            mx = jnp.where(m, mx, jnp.maximum(mx, val))
            ts = jnp.where(m, val, ts)
            return mx, ts

        init = (jnp.full((16,), -jnp.inf, jnp.float32),
                jnp.zeros((16,), jnp.float32))
        mx, ts = lax.fori_loop(0, n_classes, step, init)
        mg_v[...] = jnp.maximum(ts - mx, -10.0)
        pltpu.sync_copy(mg_v, out_hbm.at[wid])


def _margins(logits, targets):
    batch, n_classes = logits.shape
    lg = logits.reshape(16, 16, n_classes).transpose(0, 2, 1).reshape(16, 16 * n_classes)
    tg = jnp.squeeze(targets, -1).reshape(16, 16)
    mesh = plsc.VectorSubcoreMesh(core_axis_name="c", subcore_axis_name="s")
    kern = functools.partial(
        pl.kernel,
        mesh=mesh,
        out_type=jax.ShapeDtypeStruct((16, 16), jnp.float32),
        scratch_types=[
            pltpu.VMEM((16 * n_classes,), jnp.float32),
            pltpu.VMEM((16,), jnp.int32),
            pltpu.VMEM((16,), jnp.float32),
            pltpu.SemaphoreType.DMA,
        ],
    )(functools.partial(_margin_kernel, n_classes))
    return kern(lg, tg)


def kernel(w, x, logits, targets):
    b, ch, h, wd = w.shape
    batch = logits.shape[0]
    sum_sq = _mse_sum(w, x)
    margins = _margins(logits, targets)
    n_total = b * ch * h * wd
    return sum_sq / n_total + 0.5 * jnp.mean(margins.reshape(batch))
